# baseline (speedup 1.0000x reference)
import jax
import jax.numpy as jnp
from jax.experimental import pallas as pl
from jax.experimental.pallas import tpu as pltpu

EMB = 384
NEXP = 8
HID = 1536
NCLS = 1000
TOPK = 2
_EPS = 1e-5


def _layernorm(v, g, b):
    mu = jnp.mean(v, axis=-1, keepdims=True)
    var = jnp.mean((v - mu) ** 2, axis=-1, keepdims=True)
    return (v - mu) / jnp.sqrt(var + _EPS) * g + b


def _mm_t(a, w):
    return jax.lax.dot_general(
        a, w, (((1,), (1,)), ((), ())), preferred_element_type=jnp.float32
    )


def _gelu(h):
    return 0.5 * h * (1.0 + jax.lax.erf(h * (1.0 / jnp.sqrt(2.0).astype(jnp.float32))))


def _body(cls_ref, pos_ref, rw_ref, rb_ref, g1_ref, c1_ref, bv_ref, bo_ref,
          g2_ref, c2_ref, b1_ref, b2_ref, ng_ref, nb_ref, hw_ref, hb_ref,
          wv_hbm, wo_hbm, w1_hbm, w2_hbm, out_ref,
          lv_ref, ls_ref, wv_s, wo_s, w1_s, w2_s, sems):
    tokrow = cls_ref[...] + pos_ref[...]
    logits = _mm_t(tokrow, rw_ref[...]) + rb_ref[...]
    lv_ref[:, 0:NEXP] = logits
    cp = pltpu.make_async_copy(lv_ref, ls_ref, sems.at[0])
    cp.start()
    cp.wait()

    m1 = ls_ref[0, 0]
    i1 = jnp.int32(0)
    for e in range(1, NEXP):
        v = ls_ref[0, e]
        better = v > m1
        i1 = jnp.where(better, jnp.int32(e), i1)
        m1 = jnp.where(better, v, m1)
    m2 = jnp.float32(-3.0e38)
    i2 = jnp.int32(0)
    for e in range(NEXP):
        v = ls_ref[0, e]
        better = jnp.logical_and(v > m2, jnp.int32(e) != i1)
        i2 = jnp.where(better, jnp.int32(e), i2)
        m2 = jnp.where(better, v, m2)

    cps = []
    for k, e in enumerate((i1, i2)):
        cps.append(pltpu.make_async_copy(wv_hbm.at[e], wv_s.at[k], sems.at[4 * k + 1]))
        cps.append(pltpu.make_async_copy(wo_hbm.at[e], wo_s.at[k], sems.at[4 * k + 2]))
        cps.append(pltpu.make_async_copy(w1_hbm.at[e], w1_s.at[k], sems.at[4 * k + 3]))
        cps.append(pltpu.make_async_copy(w2_hbm.at[e], w2_s.at[k], sems.at[4 * k + 4]))
    for c in cps:
        c.start()
    for c in cps:
        c.wait()

    def expert_out(e, wv, wo, w1, w2):
        xn = _layernorm(tokrow, g1_ref[e], c1_ref[e])
        v = _mm_t(xn, wv) + bv_ref[e]
        attn = _mm_t(v, wo) + bo_ref[e]
        hmid = tokrow + attn
        hn = _layernorm(hmid, g2_ref[e], c2_ref[e])
        h1 = _gelu(_mm_t(hn, w1) + b1_ref[e])
        m = _mm_t(h1, w2) + b2_ref[e]
        return hmid + m

    y1 = expert_out(i1, wv_s[0], wo_s[0], w1_s[0], w2_s[0])
    y2 = expert_out(i2, wv_s[1], wo_s[1], w1_s[1], w2_s[1])
    s = (y1 + y2) * (1.0 / TOPK)
    o = _layernorm(s, ng_ref[...], nb_ref[...])
    head = _mm_t(o, hw_ref[...]) + hb_ref[...]
    out_ref[...] = jnp.broadcast_to(head, out_ref.shape)


def kernel(x, patch_W, patch_b, cls_token, pos_embed, router_W, router_b,
           ln1_g, ln1_b, Wv, bv, Wo, bo, ln2_g, ln2_b, W1, b1, W2, b2,
           norm_g, norm_b, head_W, head_b):
    Bsz = x.shape[0]
    cls2 = cls_token.reshape(1, EMB)
    pos0 = pos_embed[:, 0, :].reshape(1, EMB)

    vmem = pl.BlockSpec(memory_space=pltpu.VMEM)
    hbm = pl.BlockSpec(memory_space=pl.ANY)

    out = pl.pallas_call(
        _body,
        in_specs=[vmem] * 16 + [hbm] * 4,
        out_specs=vmem,
        out_shape=jax.ShapeDtypeStruct((Bsz, NCLS), jnp.float32),
        scratch_shapes=[
            pltpu.VMEM((1, 128), jnp.float32),
            pltpu.SMEM((1, 128), jnp.float32),
            pltpu.VMEM((TOPK, EMB, EMB), jnp.float32),
            pltpu.VMEM((TOPK, EMB, EMB), jnp.float32),
            pltpu.VMEM((TOPK, HID, EMB), jnp.float32),
            pltpu.VMEM((TOPK, EMB, HID), jnp.float32),
            pltpu.SemaphoreType.DMA((9,)),
        ],
    )(cls2, pos0, router_W, router_b.reshape(1, NEXP),
      ln1_g.reshape(NEXP, 1, EMB), ln1_b.reshape(NEXP, 1, EMB),
      bv.reshape(NEXP, 1, EMB), bo.reshape(NEXP, 1, EMB),
      ln2_g.reshape(NEXP, 1, EMB), ln2_b.reshape(NEXP, 1, EMB),
      b1.reshape(NEXP, 1, HID), b2.reshape(NEXP, 1, EMB),
      norm_g.reshape(1, EMB), norm_b.reshape(1, EMB),
      head_W, head_b.reshape(1, NCLS),
      Wv, Wo, W1, W2)
    return out
